# baseline (speedup 1.0000x reference)
import functools

import jax
import jax.numpy as jnp
from jax import lax
from jax.experimental import pallas as pl
from jax.experimental.pallas import tpu as pltpu
from jax.experimental.pallas import tpu_sc as plsc

HID = 64
L = 200
LP = 208
NW = 32
EPS = 1e-12
CHUNK = 104


def _sc_pool(xp_flat, W, n_rows):
    rows_per_w = n_rows // NW
    idx_words = rows_per_w * LP
    mesh = plsc.VectorSubcoreMesh(core_axis_name="c", subcore_axis_name="s")

    @functools.partial(
        pl.kernel,
        out_type=(
            jax.ShapeDtypeStruct((n_rows * HID,), jnp.float32),
            jax.ShapeDtypeStruct((n_rows * 16,), jnp.float32),
        ),
        mesh=mesh,
        scratch_types=[
            pltpu.VMEM((idx_words,), jnp.int32),
            pltpu.VMEM((LP, HID), jnp.float32),
            pltpu.VMEM((rows_per_w * HID,), jnp.float32),
            pltpu.VMEM((rows_per_w * 16,), jnp.float32),
            pltpu.SemaphoreType.DMA,
        ],
        compiler_params=pltpu.CompilerParams(use_tc_tiling_on_sc=False),
    )
    def k(x_hbm, w_hbm, out_hbm, cnt_hbm, idx_v, buf_v, pooled_v, cnts_v, sem):
        wid = lax.axis_index("s") * 2 + lax.axis_index("c")
        base = wid * idx_words
        pltpu.sync_copy(x_hbm.at[pl.ds(base, idx_words)], idx_v)

        @pl.loop(0, rows_per_w)
        def _row(r):
            off = r * LP
            c1 = pltpu.async_copy(
                w_hbm.at[idx_v.at[pl.ds(off, CHUNK)]],
                buf_v.at[pl.ds(0, CHUNK)],
                sem,
            )
            c2 = pltpu.async_copy(
                w_hbm.at[idx_v.at[pl.ds(off + CHUNK, CHUNK)]],
                buf_v.at[pl.ds(CHUNK, CHUNK)],
                sem,
            )

            cnt = jnp.zeros((16,), jnp.float32)
            for j in range(LP // 16):
                v = idx_v[pl.ds(off + j * 16, 16)]
                cnt = cnt + jnp.where(v != 0, 1.0, 0.0).astype(jnp.float32)
            cnts_v[pl.ds(r * 16, 16)] = cnt

            c1.wait()
            c2.wait()

            zero = jnp.zeros((16,), jnp.float32)

            @pl.loop(0, LP, init_carry=(zero, zero, zero, zero), unroll=4)
            def _sum(j, carry):
                a0, a1, a2, a3 = carry
                a0 = a0 + buf_v[j, pl.ds(0, 16)]
                a1 = a1 + buf_v[j, pl.ds(16, 16)]
                a2 = a2 + buf_v[j, pl.ds(32, 16)]
                a3 = a3 + buf_v[j, pl.ds(48, 16)]
                return (a0, a1, a2, a3)

            a0, a1, a2, a3 = _sum
            ob = r * HID
            pooled_v[pl.ds(ob, 16)] = a0
            pooled_v[pl.ds(ob + 16, 16)] = a1
            pooled_v[pl.ds(ob + 32, 16)] = a2
            pooled_v[pl.ds(ob + 48, 16)] = a3

        pltpu.sync_copy(
            pooled_v, out_hbm.at[pl.ds(wid * rows_per_w * HID, rows_per_w * HID)]
        )
        pltpu.sync_copy(
            cnts_v, cnt_hbm.at[pl.ds(wid * rows_per_w * 16, rows_per_w * 16)]
        )

    return k(xp_flat, W)


def _tc_layernorm(pooled, cnts, gamma, beta):
    def body(p_ref, c_ref, g_ref, b_ref, o_ref):
        cnt = jnp.sum(c_ref[...], axis=1, keepdims=True)
        x = p_ref[...] / cnt
        mu = jnp.mean(x, axis=1, keepdims=True)
        d = x - mu
        var = jnp.mean(d * d, axis=1, keepdims=True)
        o_ref[...] = d * lax.rsqrt(var + EPS) * g_ref[...] + b_ref[...]

    return pl.pallas_call(
        body,
        out_shape=jax.ShapeDtypeStruct(pooled.shape, jnp.float32),
    )(pooled, cnts, gamma.reshape(1, HID), beta.reshape(1, HID))


def kernel(x_s, x_t, W, gamma, beta):
    B = x_s.shape[0]
    x = jnp.concatenate([x_s, x_t], axis=0).astype(jnp.int32)
    xp = jnp.pad(x, ((0, 0), (0, LP - L))).reshape(-1)
    pooled_f, cnts_f = _sc_pool(xp, W, 2 * B)
    pooled = pooled_f.reshape(2 * B, HID)
    cnts = cnts_f.reshape(2 * B, 16)
    out = _tc_layernorm(pooled, cnts, gamma, beta)
    return out[:B], out[B:]


# speedup vs baseline: 2.6915x; 2.6915x over previous
import functools

import jax
import jax.numpy as jnp
from jax import lax
from jax.experimental import pallas as pl
from jax.experimental.pallas import tpu as pltpu
from jax.experimental.pallas import tpu_sc as plsc

HID = 64
L = 200
EPS = 1e-12
NBUF = 4
C1, C2 = 104, 96


def _sc_pool(xs_flat, xt_flat, W, n_side):
    rows_per_w = n_side // 16
    idx_words = rows_per_w * L
    mesh = plsc.VectorSubcoreMesh(core_axis_name="c", subcore_axis_name="s")

    @functools.partial(
        pl.kernel,
        out_type=(
            jax.ShapeDtypeStruct((n_side * HID,), jnp.float32),
            jax.ShapeDtypeStruct((n_side * HID,), jnp.float32),
            jax.ShapeDtypeStruct((n_side * 16,), jnp.float32),
            jax.ShapeDtypeStruct((n_side * 16,), jnp.float32),
        ),
        mesh=mesh,
        scratch_types=[
            pltpu.VMEM((idx_words + 16,), jnp.int32),
            [pltpu.VMEM((L, HID), jnp.float32) for _ in range(NBUF)],
            pltpu.VMEM((rows_per_w * HID,), jnp.float32),
            pltpu.VMEM((rows_per_w * 16,), jnp.float32),
            [pltpu.SemaphoreType.DMA for _ in range(NBUF)],
        ],
        compiler_params=pltpu.CompilerParams(use_tc_tiling_on_sc=False),
    )
    def k(xs_hbm, xt_hbm, w_hbm, os_hbm, ot_hbm, cs_hbm, ct_hbm,
          idx_v, bufs, pooled_v, cnts_v, sems):
        wid = lax.axis_index("s") * 2 + lax.axis_index("c")

        def fire(r, buf, sem):
            off = r * L
            pltpu.async_copy(
                w_hbm.at[idx_v.at[pl.ds(off, C1)]], buf.at[pl.ds(0, C1)], sem
            )
            pltpu.async_copy(
                w_hbm.at[idx_v.at[pl.ds(off + C1, C2)]],
                buf.at[pl.ds(C1, C2)],
                sem,
            )

        def wait_buf(buf, sem):
            pltpu.make_async_copy(w_hbm.at[pl.ds(0, L)], buf, sem).wait()

        lane = lax.iota(jnp.int32, 16)

        def process(r, buf):
            off = r * L
            cnt = jnp.zeros((16,), jnp.float32)
            for j in range(L // 16):
                v = idx_v[pl.ds(off + j * 16, 16)]
                cnt = cnt + jnp.where(v != 0, 1.0, 0.0).astype(jnp.float32)
            v = idx_v[pl.ds(off + (L // 16) * 16, 16)]
            tail_ok = (v != 0) & (lane < L % 16)
            cnt = cnt + jnp.where(tail_ok, 1.0, 0.0).astype(jnp.float32)
            cnts_v[pl.ds(r * 16, 16)] = cnt

            zero = jnp.zeros((16,), jnp.float32)

            @pl.loop(0, L, init_carry=(zero, zero, zero, zero), unroll=4)
            def _sum(j, carry):
                a0, a1, a2, a3 = carry
                a0 = a0 + buf[j, pl.ds(0, 16)]
                a1 = a1 + buf[j, pl.ds(16, 16)]
                a2 = a2 + buf[j, pl.ds(32, 16)]
                a3 = a3 + buf[j, pl.ds(48, 16)]
                return (a0, a1, a2, a3)

            a0, a1, a2, a3 = _sum
            ob = r * HID
            pooled_v[pl.ds(ob, 16)] = a0
            pooled_v[pl.ds(ob + 16, 16)] = a1
            pooled_v[pl.ds(ob + 32, 16)] = a2
            pooled_v[pl.ds(ob + 48, 16)] = a3

        def side(x_hbm, out_hbm, cnt_hbm, sw):
            pltpu.sync_copy(
                x_hbm.at[pl.ds(sw * idx_words, idx_words)],
                idx_v.at[pl.ds(0, idx_words)],
            )
            for b in range(NBUF):
                fire(b, bufs[b], sems[b])

            @pl.loop(0, rows_per_w // NBUF)
            def _outer(g):
                for b in range(NBUF):
                    r = g * NBUF + b
                    wait_buf(bufs[b], sems[b])
                    process(r, bufs[b])

                    @pl.when(r + NBUF < rows_per_w)
                    def _():
                        fire(r + NBUF, bufs[b], sems[b])

            pltpu.sync_copy(
                pooled_v,
                out_hbm.at[pl.ds(sw * rows_per_w * HID, rows_per_w * HID)],
            )
            pltpu.sync_copy(
                cnts_v, cnt_hbm.at[pl.ds(sw * rows_per_w * 16, rows_per_w * 16)]
            )

        @pl.when(wid < 16)
        def _():
            side(xs_hbm, os_hbm, cs_hbm, wid)

        @pl.when(wid >= 16)
        def _():
            side(xt_hbm, ot_hbm, ct_hbm, wid - 16)

    return k(xs_flat, xt_flat, W)


def _tc_layernorm(pooled_s, pooled_t, cnts_s, cnts_t, gamma, beta):
    def body(ps_ref, pt_ref, cs_ref, ct_ref, g_ref, b_ref, os_ref, ot_ref):
        g = g_ref[...]
        b = b_ref[...]
        for p_ref, c_ref, o_ref in (
            (ps_ref, cs_ref, os_ref),
            (pt_ref, ct_ref, ot_ref),
        ):
            cnt = jnp.sum(c_ref[...], axis=1, keepdims=True)
            x = p_ref[...] / cnt
            mu = jnp.mean(x, axis=1, keepdims=True)
            d = x - mu
            var = jnp.mean(d * d, axis=1, keepdims=True)
            o_ref[...] = d * lax.rsqrt(var + EPS) * g + b

    n = pooled_s.shape[0]
    return pl.pallas_call(
        body,
        out_shape=(
            jax.ShapeDtypeStruct((n, HID), jnp.float32),
            jax.ShapeDtypeStruct((n, HID), jnp.float32),
        ),
    )(pooled_s, pooled_t, cnts_s, cnts_t,
      gamma.reshape(1, HID), beta.reshape(1, HID))


def kernel(x_s, x_t, W, gamma, beta):
    B = x_s.shape[0]
    xs = x_s.astype(jnp.int32).reshape(-1)
    xt = x_t.astype(jnp.int32).reshape(-1)
    ps, pt, cs, ct = _sc_pool(xs, xt, W, B)
    out_s, out_t = _tc_layernorm(
        ps.reshape(B, HID),
        pt.reshape(B, HID),
        cs.reshape(B, 16),
        ct.reshape(B, 16),
        gamma,
        beta,
    )
    return out_s, out_t
